# Initial kernel scaffold; baseline (speedup 1.0000x reference)
import jax
import jax.numpy as jnp
from jax.experimental import pallas as pl


def kernel(logits, targets, BCE_L):
    raise NotImplementedError("write your pallas kernel here")


# speedup vs baseline: 1.2947x; 1.2947x over previous
import functools
import math

import jax
import jax.numpy as jnp
from jax import lax
from jax.experimental import pallas as pl
from jax.experimental.pallas import tpu as pltpu
from jax.experimental.pallas import tpu_sc as plsc

_K = 20
_NEG_INF = float("-inf")
_BIG_I32 = 2**31 - 1

_R = 128
_N = 100000
_BLK = 2048
_NB = math.ceil(_N / _BLK)
_NPAD = _NB * _BLK
_SEG = 512
_NSEG = _NPAD // _SEG
_SEGPAD = 200
_SPB = _BLK // _SEG
_CAP = 128


def _phase_a_body(l_ref, t_ref, bce_ref, tau_ref, segmax_ref, fold_ref,
                  acc_ref):
    j = pl.program_id(0)

    @pl.when(j == 0)
    def _init():
        fold_ref[...] = jnp.full_like(fold_ref, _NEG_INF)
        acc_ref[0, 0] = jnp.float32(0.0)

    l = l_ref[...]
    t = t_ref[...]
    col = j * _BLK + lax.broadcasted_iota(jnp.int32, (_R, _BLK), 1)
    valid = col < _N

    bce = jnp.maximum(l, 0.0) - l * t + jnp.log1p(jnp.exp(-jnp.abs(l)))
    bce = jnp.where(valid, bce, 0.0)
    acc_ref[0, 0] += jnp.sum(bce)

    lv = jnp.where(valid, l, _NEG_INF)
    fold_ref[...] = jnp.maximum(fold_ref[...], lv)

    segs = [
        jnp.max(lv[:, k * _SEG:(k + 1) * _SEG], axis=1, keepdims=True)
        for k in range(_SPB)
    ]
    segmax_ref[...] = jnp.concatenate(segs, axis=1)[None, :, :]

    @pl.when(j == _NB - 1)
    def _finish():
        ids = lax.broadcasted_iota(jnp.int32, (_R, _BLK), 1)
        buf = fold_ref[...]
        m = None
        for _ in range(_K):
            m = jnp.max(buf, axis=1, keepdims=True)
            ci = jnp.where(buf == m, ids, _BIG_I32)
            si = jnp.min(ci, axis=1, keepdims=True)
            buf = jnp.where(ids == si, _NEG_INF, buf)
        tau_ref[...] = m
        bce_ref[0, 0] = acc_ref[0, 0]


def _phase_a(logits_pad, targets_pad):
    return pl.pallas_call(
        _phase_a_body,
        grid=(_NB,),
        in_specs=[
            pl.BlockSpec((_R, _BLK), lambda j: (0, j)),
            pl.BlockSpec((_R, _BLK), lambda j: (0, j)),
        ],
        out_specs=[
            pl.BlockSpec(memory_space=pltpu.SMEM),
            pl.BlockSpec((_R, 1), lambda j: (0, 0)),
            pl.BlockSpec((1, _R, _SPB), lambda j: (j, 0, 0)),
        ],
        out_shape=[
            jax.ShapeDtypeStruct((1, 1), jnp.float32),
            jax.ShapeDtypeStruct((_R, 1), jnp.float32),
            jax.ShapeDtypeStruct((_NB, _R, _SPB), jnp.float32),
        ],
        scratch_shapes=[
            pltpu.VMEM((_R, _BLK), jnp.float32),
            pltpu.SMEM((1, 1), jnp.float32),
        ],
        compiler_params=pltpu.CompilerParams(
            dimension_semantics=("arbitrary",),
        ),
    )(logits_pad, targets_pad)


_WL = _NSEG + 48


def _phase_b_body(l_ref, t_ref, tau_ref, segmax_ref,
                  cv_ref, ct_ref, ci_ref,
                  tau_v, segmax_v, wl_v, seg_l, seg_t, cv_v, ct_v, ci_v):
    info = plsc.get_sparse_core_info()
    nc = info.num_cores
    wid = lax.axis_index("s") * nc + lax.axis_index("c")
    rows_per_w = _R // (nc * info.num_subcores)

    pltpu.sync_copy(tau_ref, tau_v.at[pl.ds(0, _R)])
    pltpu.sync_copy(segmax_ref, segmax_v.at[pl.ds(0, _NB * _R * _SPB)])
    lane = lax.iota(jnp.int32, 16)

    for k in range(rows_per_w):
        r = wid * rows_per_w + k
        tau_vec = jnp.full((16,), tau_v[pl.ds(r, 16)][0], jnp.float32)

        for i in range(_CAP // 16):
            cv_v[pl.ds(i * 16, 16)] = jnp.full((16,), _NEG_INF, jnp.float32)
            ct_v[pl.ds(i * 16, 16)] = jnp.zeros((16,), jnp.float32)
            ci_v[pl.ds(i * 16, 16)] = jnp.full((16,), _BIG_I32, jnp.int32)

        wptr = jnp.int32(0)
        for jb in range(_NB):
            m16 = segmax_v[pl.ds(jb * (_R * _SPB) + r * _SPB, 16)]
            act = (m16 >= tau_vec) & (lane < _SPB)
            s16 = jb * _SPB + lane
            pos = plsc.cumsum(act.astype(jnp.int32))
            widx = jnp.where(act, wptr + pos - 1, _NSEG + 16 + lane)
            plsc.store_scatter(wl_v, [widx], s16)
            wptr = wptr + pos[15]

        def wl_body(i, ptr):
            s = wl_v[pl.ds(i, 16)][0]
            base = r * _NPAD + s * _SEG
            pltpu.sync_copy(l_ref.at[pl.ds(base, _SEG)], seg_l)
            pltpu.sync_copy(t_ref.at[pl.ds(base, _SEG)], seg_t)
            for v in range(_SEG // 16):
                lv = seg_l[pl.ds(v * 16, 16)]
                gi = s * _SEG + v * 16 + lane
                msk = (lv >= tau_vec) & (gi < _N)
                cnt = jnp.sum(msk.astype(jnp.int32))

                def emit(pp):
                    tv = seg_t[pl.ds(v * 16, 16)]
                    pos = plsc.cumsum(msk.astype(jnp.int32))
                    cidx = jnp.where(
                        msk,
                        jnp.minimum(pp + pos - 1, _CAP + 15),
                        _CAP + lane,
                    )
                    plsc.store_scatter(cv_v, [cidx], lv)
                    plsc.store_scatter(ct_v, [cidx], tv)
                    plsc.store_scatter(ci_v, [cidx], gi)
                    return pp + cnt

                ptr = lax.cond(cnt > 0, emit, lambda pp: pp, ptr)
            return ptr

        lax.fori_loop(0, wptr, wl_body, jnp.int32(0))

        pltpu.sync_copy(cv_v.at[pl.ds(0, _CAP)], cv_ref.at[pl.ds(r * _CAP, _CAP)])
        pltpu.sync_copy(ct_v.at[pl.ds(0, _CAP)], ct_ref.at[pl.ds(r * _CAP, _CAP)])
        pltpu.sync_copy(ci_v.at[pl.ds(0, _CAP)], ci_ref.at[pl.ds(r * _CAP, _CAP)])


def _phase_b(l_flat, t_flat, tau_flat, segmax_flat):
    mesh = plsc.VectorSubcoreMesh(core_axis_name="c", subcore_axis_name="s")
    kb = functools.partial(
        pl.kernel,
        mesh=mesh,
        compiler_params=pltpu.CompilerParams(needs_layout_passes=False),
        out_type=[
            jax.ShapeDtypeStruct((_R * _CAP,), jnp.float32),
            jax.ShapeDtypeStruct((_R * _CAP,), jnp.float32),
            jax.ShapeDtypeStruct((_R * _CAP,), jnp.int32),
        ],
        scratch_types=[
            pltpu.VMEM((_R + 16,), jnp.float32),
            pltpu.VMEM((_NB * _R * _SPB + 16,), jnp.float32),
            pltpu.VMEM((_WL,), jnp.int32),
            pltpu.VMEM((_SEG,), jnp.float32),
            pltpu.VMEM((_SEG,), jnp.float32),
            pltpu.VMEM((_CAP + 16,), jnp.float32),
            pltpu.VMEM((_CAP + 16,), jnp.float32),
            pltpu.VMEM((_CAP + 16,), jnp.int32),
        ],
    )(_phase_b_body)
    return kb(l_flat, t_flat, tau_flat, segmax_flat)


def _phase_c_body(cv_ref, ct_ref, ci_ref, bce_ref, bcel_ref, out_ref):
    buf = cv_ref[...]
    ibuf = ci_ref[...]
    tbuf = ct_ref[...]

    picks_v, picks_t = [], []
    for _ in range(_K):
        m = jnp.max(buf, axis=1, keepdims=True)
        ci = jnp.where(buf == m, ibuf, _BIG_I32)
        si = jnp.min(ci, axis=1, keepdims=True)
        sel = ibuf == si
        tm = jnp.sum(jnp.where(sel, tbuf, 0.0), axis=1, keepdims=True)
        picks_v.append(m)
        picks_t.append(tm)
        buf = jnp.where(sel, _NEG_INF, buf)

    vs = jnp.concatenate(picks_v, axis=1)
    ts = jnp.concatenate(picks_t, axis=1)
    probs = jax.nn.sigmoid(vs)
    logp = jnp.maximum(jnp.log(probs), -100.0)
    log1mp = jnp.maximum(jnp.log(1.0 - probs), -100.0)
    row_bce = -jnp.sum(ts * logp + (1.0 - ts) * log1mp, axis=1) / _K
    top_loss = jnp.sum(row_bce) / _R
    bce_mean = bce_ref[0, 0] / (_R * _N)
    out_ref[0, 0] = bce_mean + bcel_ref[0, 0] * top_loss


def _phase_c(cv, ct, ci, bce_sum, bcel):
    return pl.pallas_call(
        _phase_c_body,
        in_specs=[
            pl.BlockSpec((_R, _CAP), lambda: (0, 0)),
            pl.BlockSpec((_R, _CAP), lambda: (0, 0)),
            pl.BlockSpec((_R, _CAP), lambda: (0, 0)),
            pl.BlockSpec(memory_space=pltpu.SMEM),
            pl.BlockSpec(memory_space=pltpu.SMEM),
        ],
        out_specs=pl.BlockSpec(memory_space=pltpu.SMEM),
        out_shape=jax.ShapeDtypeStruct((1, 1), jnp.float32),
    )(cv, ct, ci, bce_sum, bcel)


@jax.jit
def kernel(logits, targets, BCE_L):
    pad = ((0, 0), (0, _NPAD - _N))
    logits_pad = jnp.pad(logits, pad)
    targets_pad = jnp.pad(targets, pad)
    bcel = jnp.reshape(BCE_L, (1, 1)).astype(jnp.float32)

    bce_sum, tau, segmax = _phase_a(logits_pad, targets_pad)

    cv, ct, ci = _phase_b(
        jnp.reshape(logits_pad, (-1,)),
        jnp.reshape(targets_pad, (-1,)),
        jnp.reshape(tau, (-1,)),
        jnp.reshape(segmax, (-1,)),
    )

    out = _phase_c(
        jnp.reshape(cv, (_R, _CAP)),
        jnp.reshape(ct, (_R, _CAP)),
        jnp.reshape(ci, (_R, _CAP)),
        bce_sum,
        bcel,
    )
    return out[0, 0]


# speedup vs baseline: 1.5330x; 1.1841x over previous
import functools
import math

import jax
import jax.numpy as jnp
from jax import lax
from jax.experimental import pallas as pl
from jax.experimental.pallas import tpu as pltpu
from jax.experimental.pallas import tpu_sc as plsc

_K = 20
_NEG_INF = float("-inf")
_BIG_I32 = 2**31 - 1

_R = 128
_N = 100000
_BLK = 2048
_NB = math.ceil(_N / _BLK)
_SEG = 512
_NSEG = _NB * _BLK // _SEG
_SPB = _BLK // _SEG
_CAP = 128
_LASTBASE = _N - _SEG
_WL = _NSEG + 48
_GK = 64


def _phase_a_body(l_ref, t_ref, bce_ref, fold_out_ref, segmax_ref, fold_ref,
                  acc_ref):
    j = pl.program_id(0)

    @pl.when(j == 0)
    def _init():
        fold_ref[...] = jnp.full_like(fold_ref, _NEG_INF)
        acc_ref[0, 0] = jnp.float32(0.0)

    l = l_ref[...]
    t = t_ref[...]
    col = j * _BLK + lax.broadcasted_iota(jnp.int32, (_R, _BLK), 1)
    valid = col < _N

    bce = jnp.maximum(l, 0.0) - l * t + jnp.log1p(jnp.exp(-jnp.abs(l)))
    bce = jnp.where(valid, bce, 0.0)
    acc_ref[0, 0] += jnp.sum(bce)

    lv = jnp.where(valid, l, _NEG_INF)
    fold_ref[...] = jnp.maximum(fold_ref[...], lv)

    segs = [
        jnp.max(lv[:, k * _SEG:(k + 1) * _SEG], axis=1, keepdims=True)
        for k in range(_SPB)
    ]
    segmax_ref[...] = jnp.concatenate(segs, axis=1)[None, :, :]

    @pl.when(j == _NB - 1)
    def _finish():
        fold_out_ref[...] = fold_ref[...]
        bce_ref[0, 0] = acc_ref[0, 0]


def _phase_a(logits, targets):
    return pl.pallas_call(
        _phase_a_body,
        grid=(_NB,),
        in_specs=[
            pl.BlockSpec((_R, _BLK), lambda j: (0, j)),
            pl.BlockSpec((_R, _BLK), lambda j: (0, j)),
        ],
        out_specs=[
            pl.BlockSpec(memory_space=pltpu.SMEM),
            pl.BlockSpec((_R, _BLK), lambda j: (0, 0)),
            pl.BlockSpec((1, _R, _SPB), lambda j: (j, 0, 0)),
        ],
        out_shape=[
            jax.ShapeDtypeStruct((1, 1), jnp.float32),
            jax.ShapeDtypeStruct((_R, _BLK), jnp.float32),
            jax.ShapeDtypeStruct((_NB, _R, _SPB), jnp.float32),
        ],
        scratch_shapes=[
            pltpu.VMEM((_R, _BLK), jnp.float32),
            pltpu.SMEM((1, 1), jnp.float32),
        ],
        compiler_params=pltpu.CompilerParams(
            dimension_semantics=("arbitrary",),
        ),
    )(logits, targets)


def _phase_a2_body(fold_ref, tau_ref):
    ids = lax.broadcasted_iota(jnp.int32, (_R, _BLK), 1)
    buf = fold_ref[...]
    m = None
    for _ in range(_K):
        m = jnp.max(buf, axis=1, keepdims=True)
        ci = jnp.where(buf == m, ids, _BIG_I32)
        si = jnp.min(ci, axis=1, keepdims=True)
        buf = jnp.where(ids == si, _NEG_INF, buf)
    tau_ref[...] = m


def _phase_a2(fold):
    return pl.pallas_call(
        _phase_a2_body,
        in_specs=[pl.BlockSpec((_R, _BLK), lambda: (0, 0))],
        out_specs=pl.BlockSpec((_R, 1), lambda: (0, 0)),
        out_shape=jax.ShapeDtypeStruct((_R, 1), jnp.float32),
    )(fold)


def _phase_b_body(l_ref, t_ref, tau_ref, segmax_ref,
                  cv_ref, ct_ref, ci_ref,
                  tau_v, segmax_v, wl_v, stage_l, stage_t,
                  cv_v, ct_v, ci_v, sem_l, sem_t):
    info = plsc.get_sparse_core_info()
    nc = info.num_cores
    wid = lax.axis_index("s") * nc + lax.axis_index("c")
    rows_per_w = _R // (nc * info.num_subcores)

    pltpu.sync_copy(tau_ref, tau_v.at[pl.ds(0, _R)])
    pltpu.sync_copy(segmax_ref, segmax_v.at[pl.ds(0, _NB * _R * _SPB)])
    lane = lax.iota(jnp.int32, 16)

    for k in range(rows_per_w):
        r = wid * rows_per_w + k
        tau_vec = jnp.full((16,), tau_v[pl.ds(r, 16)][0], jnp.float32)

        for i in range(_CAP // 16):
            cv_v[pl.ds(i * 16, 16)] = jnp.full((16,), _NEG_INF, jnp.float32)
            ct_v[pl.ds(i * 16, 16)] = jnp.zeros((16,), jnp.float32)
            ci_v[pl.ds(i * 16, 16)] = jnp.full((16,), _BIG_I32, jnp.int32)

        wptr = jnp.int32(0)
        for jb in range(_NB):
            m16 = segmax_v[pl.ds(jb * (_R * _SPB) + r * _SPB, 16)]
            act = (m16 >= tau_vec) & (lane < _SPB)
            s16 = jb * _SPB + lane
            pos = plsc.cumsum(act.astype(jnp.int32))
            widx = jnp.where(act, wptr + pos - 1, _NSEG + 16 + lane)
            plsc.store_scatter(wl_v, [widx], s16)
            wptr = wptr + pos[15]

        ngroups = (wptr + (_GK - 1)) // _GK

        def group_body(g, ptr):
            g0 = g * _GK
            gk = jnp.minimum(wptr - g0, _GK)

            def issue(i, _):
                s = wl_v[pl.ds(g0 + i, 16)][0]
                base = r * _N + jnp.minimum(s * _SEG, _LASTBASE)
                pltpu.make_async_copy(
                    l_ref.at[pl.ds(base, _SEG)],
                    stage_l.at[pl.ds(i * _SEG, _SEG)], sem_l).start()
                pltpu.make_async_copy(
                    t_ref.at[pl.ds(base, _SEG)],
                    stage_t.at[pl.ds(i * _SEG, _SEG)], sem_t).start()
                return jnp.int32(0)

            lax.fori_loop(0, gk, issue, jnp.int32(0))

            def drain(i, _):
                pltpu.make_async_copy(
                    l_ref.at[pl.ds(0, _SEG)],
                    stage_l.at[pl.ds(0, _SEG)], sem_l).wait()
                pltpu.make_async_copy(
                    t_ref.at[pl.ds(0, _SEG)],
                    stage_t.at[pl.ds(0, _SEG)], sem_t).wait()
                return jnp.int32(0)

            lax.fori_loop(0, gk, drain, jnp.int32(0))

            def proc(i, p):
                s = wl_v[pl.ds(g0 + i, 16)][0]
                cb = jnp.minimum(s * _SEG, _LASTBASE)
                smin = s * _SEG
                for v in range(_SEG // 16):
                    lv = stage_l[pl.ds(i * _SEG + v * 16, 16)]
                    gi = cb + v * 16 + lane
                    msk = (lv >= tau_vec) & (gi >= smin)
                    cnt = plsc.all_reduce_population_count(msk)[0]

                    def emit(pp):
                        tv = stage_t[pl.ds(i * _SEG + v * 16, 16)]
                        pos = plsc.cumsum(msk.astype(jnp.int32))
                        cidx = jnp.where(
                            msk,
                            jnp.minimum(pp + pos - 1, _CAP + 15),
                            _CAP + lane,
                        )
                        plsc.store_scatter(cv_v, [cidx], lv)
                        plsc.store_scatter(ct_v, [cidx], tv)
                        plsc.store_scatter(ci_v, [cidx], gi)
                        return pp + cnt

                    p = lax.cond(cnt > 0, emit, lambda pp: pp, p)
                return p

            return lax.fori_loop(0, gk, proc, ptr)

        lax.fori_loop(0, ngroups, group_body, jnp.int32(0))

        pltpu.sync_copy(cv_v.at[pl.ds(0, _CAP)],
                        cv_ref.at[pl.ds(r * _CAP, _CAP)])
        pltpu.sync_copy(ct_v.at[pl.ds(0, _CAP)],
                        ct_ref.at[pl.ds(r * _CAP, _CAP)])
        pltpu.sync_copy(ci_v.at[pl.ds(0, _CAP)],
                        ci_ref.at[pl.ds(r * _CAP, _CAP)])


def _phase_b(l_flat, t_flat, tau_flat, segmax_flat):
    mesh = plsc.VectorSubcoreMesh(core_axis_name="c", subcore_axis_name="s")
    kb = functools.partial(
        pl.kernel,
        mesh=mesh,
        compiler_params=pltpu.CompilerParams(needs_layout_passes=False),
        out_type=[
            jax.ShapeDtypeStruct((_R * _CAP,), jnp.float32),
            jax.ShapeDtypeStruct((_R * _CAP,), jnp.float32),
            jax.ShapeDtypeStruct((_R * _CAP,), jnp.int32),
        ],
        scratch_types=[
            pltpu.VMEM((_R + 16,), jnp.float32),
            pltpu.VMEM((_NB * _R * _SPB + 16,), jnp.float32),
            pltpu.VMEM((_WL,), jnp.int32),
            pltpu.VMEM((_GK * _SEG,), jnp.float32),
            pltpu.VMEM((_GK * _SEG,), jnp.float32),
            pltpu.VMEM((_CAP + 16,), jnp.float32),
            pltpu.VMEM((_CAP + 16,), jnp.float32),
            pltpu.VMEM((_CAP + 16,), jnp.int32),
            pltpu.SemaphoreType.DMA,
            pltpu.SemaphoreType.DMA,
        ],
    )(_phase_b_body)
    return kb(l_flat, t_flat, tau_flat, segmax_flat)


def _phase_c_body(cv_ref, ct_ref, ci_ref, bce_ref, bcel_ref, out_ref):
    buf = cv_ref[...]
    ibuf = ci_ref[...]
    tbuf = ct_ref[...]

    picks_v, picks_t = [], []
    for _ in range(_K):
        m = jnp.max(buf, axis=1, keepdims=True)
        ci = jnp.where(buf == m, ibuf, _BIG_I32)
        si = jnp.min(ci, axis=1, keepdims=True)
        sel = ibuf == si
        tm = jnp.sum(jnp.where(sel, tbuf, 0.0), axis=1, keepdims=True)
        picks_v.append(m)
        picks_t.append(tm)
        buf = jnp.where(sel, _NEG_INF, buf)

    vs = jnp.concatenate(picks_v, axis=1)
    ts = jnp.concatenate(picks_t, axis=1)
    probs = jax.nn.sigmoid(vs)
    logp = jnp.maximum(jnp.log(probs), -100.0)
    log1mp = jnp.maximum(jnp.log(1.0 - probs), -100.0)
    row_bce = -jnp.sum(ts * logp + (1.0 - ts) * log1mp, axis=1) / _K
    top_loss = jnp.sum(row_bce) / _R
    bce_mean = bce_ref[0, 0] / (_R * _N)
    out_ref[0, 0] = bce_mean + bcel_ref[0, 0] * top_loss


def _phase_c(cv, ct, ci, bce_sum, bcel):
    return pl.pallas_call(
        _phase_c_body,
        in_specs=[
            pl.BlockSpec((_R, _CAP), lambda: (0, 0)),
            pl.BlockSpec((_R, _CAP), lambda: (0, 0)),
            pl.BlockSpec((_R, _CAP), lambda: (0, 0)),
            pl.BlockSpec(memory_space=pltpu.SMEM),
            pl.BlockSpec(memory_space=pltpu.SMEM),
        ],
        out_specs=pl.BlockSpec(memory_space=pltpu.SMEM),
        out_shape=jax.ShapeDtypeStruct((1, 1), jnp.float32),
    )(cv, ct, ci, bce_sum, bcel)


@jax.jit
def kernel(logits, targets, BCE_L):
    bcel = jnp.reshape(BCE_L, (1, 1)).astype(jnp.float32)

    bce_sum, fold, segmax = _phase_a(logits, targets)
    tau = _phase_a2(fold)

    cv, ct, ci = _phase_b(
        jnp.reshape(logits, (-1,)),
        jnp.reshape(targets, (-1,)),
        jnp.reshape(tau, (-1,)),
        jnp.reshape(segmax, (-1,)),
    )

    out = _phase_c(
        jnp.reshape(cv, (_R, _CAP)),
        jnp.reshape(ct, (_R, _CAP)),
        jnp.reshape(ci, (_R, _CAP)),
        bce_sum,
        bcel,
    )
    return out[0, 0]


# speedup vs baseline: 2.8410x; 1.8532x over previous
import functools
import math

import jax
import jax.numpy as jnp
from jax import lax
from jax.experimental import pallas as pl
from jax.experimental.pallas import tpu as pltpu
from jax.experimental.pallas import tpu_sc as plsc

_K = 20
_NEG_INF = float("-inf")
_BIG_I32 = 2**31 - 1

_R = 128
_N = 100000
_BLK = 2048
_NB = math.ceil(_N / _BLK)
_SEG = 128
_SPB = _BLK // _SEG
_NSEGPAD = _NB * _SPB
_NT = _N // _SEG
_TAIL = _NT * _SEG
_NTAIL = _N - _TAIL
_CAP = 128
_WL = _NT + 48
_GT = 48


def _phase_a_body(l_ref, t_ref, bce_ref, segmax_ref, acc_ref):
    j = pl.program_id(0)

    @pl.when(j == 0)
    def _init():
        acc_ref[0, 0] = jnp.float32(0.0)

    l = l_ref[...]
    t = t_ref[...]
    col = j * _BLK + lax.broadcasted_iota(jnp.int32, (_R, _BLK), 1)
    valid = col < _N

    bce = jnp.maximum(l, 0.0) - l * t + jnp.log1p(jnp.exp(-jnp.abs(l)))
    bce = jnp.where(valid, bce, 0.0)
    acc_ref[0, 0] += jnp.sum(bce)

    lv = jnp.where(valid, l, _NEG_INF)
    segs = [
        jnp.max(lv[:, k * _SEG:(k + 1) * _SEG], axis=1, keepdims=True)
        for k in range(_SPB)
    ]
    segmax_ref[...] = jnp.concatenate(segs, axis=1)[None, :, :]

    @pl.when(j == _NB - 1)
    def _finish():
        bce_ref[0, 0] = acc_ref[0, 0]


def _phase_a(logits, targets):
    return pl.pallas_call(
        _phase_a_body,
        grid=(_NB,),
        in_specs=[
            pl.BlockSpec((_R, _BLK), lambda j: (0, j)),
            pl.BlockSpec((_R, _BLK), lambda j: (0, j)),
        ],
        out_specs=[
            pl.BlockSpec(memory_space=pltpu.SMEM),
            pl.BlockSpec((1, _R, _SPB), lambda j: (j, 0, 0)),
        ],
        out_shape=[
            jax.ShapeDtypeStruct((1, 1), jnp.float32),
            jax.ShapeDtypeStruct((_NB, _R, _SPB), jnp.float32),
        ],
        scratch_shapes=[
            pltpu.SMEM((1, 1), jnp.float32),
        ],
        compiler_params=pltpu.CompilerParams(
            dimension_semantics=("arbitrary",),
        ),
    )(logits, targets)


def _phase_a2_body(segmax_ref, tau_ref):
    ids = lax.broadcasted_iota(jnp.int32, (_R, _NSEGPAD), 1)
    buf = segmax_ref[...]
    m = None
    for _ in range(_K):
        m = jnp.max(buf, axis=1, keepdims=True)
        ci = jnp.where(buf == m, ids, _BIG_I32)
        si = jnp.min(ci, axis=1, keepdims=True)
        buf = jnp.where(ids == si, _NEG_INF, buf)
    tau_ref[...] = m


def _phase_a2(segmax_t):
    return pl.pallas_call(
        _phase_a2_body,
        in_specs=[pl.BlockSpec((_R, _NSEGPAD), lambda: (0, 0))],
        out_specs=pl.BlockSpec((_R, 1), lambda: (0, 0)),
        out_shape=jax.ShapeDtypeStruct((_R, 1), jnp.float32),
    )(segmax_t)


def _phase_b_body(l_ref, t_ref, tau_ref, segmax_ref,
                  cv_ref, ct_ref, ci_ref,
                  tau_v, sm4, wl_v, stage_l, stage_t,
                  cv_v, ct_v, ci_v, sem_l, sem_t, sem_s):
    info = plsc.get_sparse_core_info()
    nc = info.num_cores
    wid = lax.axis_index("s") * nc + lax.axis_index("c")
    rows_per_w = _R // (nc * info.num_subcores)

    pltpu.sync_copy(tau_ref, tau_v.at[pl.ds(0, _R)])
    lane = lax.iota(jnp.int32, 16)

    for j in range(_NB):
        pltpu.make_async_copy(
            segmax_ref.at[pl.ds(j * (_R * _SPB) + wid * (rows_per_w * _SPB),
                                rows_per_w * _SPB)],
            sm4.at[pl.ds(j * (rows_per_w * _SPB), rows_per_w * _SPB)],
            sem_s).start()
    for j in range(_NB):
        pltpu.make_async_copy(
            segmax_ref.at[pl.ds(0, rows_per_w * _SPB)],
            sm4.at[pl.ds(0, rows_per_w * _SPB)], sem_s).wait()

    r8 = (wid // 2) * 8

    for k in range(rows_per_w):
        r = wid * rows_per_w + k
        rr = (wid % 2) * 4 + k
        tau_vec = jnp.full((16,), tau_v[pl.ds(r, 16)][0], jnp.float32)

        for i in range(_CAP // 16):
            cv_v[pl.ds(i * 16, 16)] = jnp.full((16,), _NEG_INF, jnp.float32)
            ct_v[pl.ds(i * 16, 16)] = jnp.zeros((16,), jnp.float32)
            ci_v[pl.ds(i * 16, 16)] = jnp.full((16,), _BIG_I32, jnp.int32)

        wptr = jnp.int32(0)
        for j in range(_NB):
            m16 = sm4[pl.ds(j * (rows_per_w * _SPB) + k * _SPB, 16)]
            act = m16 >= tau_vec
            if (j + 1) * _SPB > _NT:
                act = act & (lane < _NT - j * _SPB)
            s16 = j * _SPB + lane
            pos = plsc.cumsum(act.astype(jnp.int32))
            widx = jnp.where(act, wptr + pos - 1, _NT + 16 + lane)
            plsc.store_scatter(wl_v, [widx], s16)
            wptr = wptr + pos[15]

        ngroups = (wptr + (_GT - 1)) // _GT

        def group_body(g, ptr):
            g0 = g * _GT
            gk = jnp.minimum(wptr - g0, _GT)

            def issue(i, _):
                s = wl_v[pl.ds(g0 + i, 16)][0]
                pltpu.make_async_copy(
                    l_ref.at[pl.ds(r8, 8), pl.ds(s * _SEG, _SEG)],
                    stage_l.at[i], sem_l).start()
                pltpu.make_async_copy(
                    t_ref.at[pl.ds(r8, 8), pl.ds(s * _SEG, _SEG)],
                    stage_t.at[i], sem_t).start()
                return jnp.int32(0)

            lax.fori_loop(0, gk, issue, jnp.int32(0))

            def drain(i, _):
                pltpu.make_async_copy(
                    l_ref.at[pl.ds(0, 8), pl.ds(0, _SEG)],
                    stage_l.at[0], sem_l).wait()
                pltpu.make_async_copy(
                    t_ref.at[pl.ds(0, 8), pl.ds(0, _SEG)],
                    stage_t.at[0], sem_t).wait()
                return jnp.int32(0)

            lax.fori_loop(0, gk, drain, jnp.int32(0))

            def proc(i, p):
                s = wl_v[pl.ds(g0 + i, 16)][0]
                cb = s * _SEG
                for v in range(_SEG // 16):
                    lv = stage_l[i, rr, pl.ds(v * 16, 16)]
                    gi = cb + v * 16 + lane
                    msk = lv >= tau_vec
                    cnt = plsc.all_reduce_population_count(msk)[0]

                    def emit(pp):
                        tv = stage_t[i, rr, pl.ds(v * 16, 16)]
                        pos = plsc.cumsum(msk.astype(jnp.int32))
                        cidx = jnp.where(
                            msk,
                            jnp.minimum(pp + pos - 1, _CAP + 15),
                            _CAP + lane,
                        )
                        plsc.store_scatter(cv_v, [cidx], lv)
                        plsc.store_scatter(ct_v, [cidx], tv)
                        plsc.store_scatter(ci_v, [cidx], gi)
                        return pp + cnt

                    p = lax.cond(cnt > 0, emit, lambda pp: pp, p)
                return p

            return lax.fori_loop(0, gk, proc, ptr)

        lax.fori_loop(0, ngroups, group_body, jnp.int32(0))

        pltpu.sync_copy(cv_v.at[pl.ds(0, _CAP)],
                        cv_ref.at[pl.ds(r * _CAP, _CAP)])
        pltpu.sync_copy(ct_v.at[pl.ds(0, _CAP)],
                        ct_ref.at[pl.ds(r * _CAP, _CAP)])
        pltpu.sync_copy(ci_v.at[pl.ds(0, _CAP)],
                        ci_ref.at[pl.ds(r * _CAP, _CAP)])


def _phase_b(logits, targets, tau_flat, segmax_flat):
    mesh = plsc.VectorSubcoreMesh(core_axis_name="c", subcore_axis_name="s")
    kb = functools.partial(
        pl.kernel,
        mesh=mesh,
        compiler_params=pltpu.CompilerParams(
            needs_layout_passes=False, use_tc_tiling_on_sc=True),
        out_type=[
            jax.ShapeDtypeStruct((_R * _CAP,), jnp.float32),
            jax.ShapeDtypeStruct((_R * _CAP,), jnp.float32),
            jax.ShapeDtypeStruct((_R * _CAP,), jnp.int32),
        ],
        scratch_types=[
            pltpu.VMEM((_R + 16,), jnp.float32),
            pltpu.VMEM((_NB * 4 * _SPB,), jnp.float32),
            pltpu.VMEM((_WL,), jnp.int32),
            pltpu.VMEM((_GT, 8, _SEG), jnp.float32),
            pltpu.VMEM((_GT, 8, _SEG), jnp.float32),
            pltpu.VMEM((_CAP + 16,), jnp.float32),
            pltpu.VMEM((_CAP + 16,), jnp.float32),
            pltpu.VMEM((_CAP + 16,), jnp.int32),
            pltpu.SemaphoreType.DMA,
            pltpu.SemaphoreType.DMA,
            pltpu.SemaphoreType.DMA,
        ],
    )(_phase_b_body)
    return kb(logits, targets, tau_flat, segmax_flat)


def _phase_c_body(cv_ref, ct_ref, ci_ref, tl_ref, tt_ref, bce_ref, bcel_ref,
                  out_ref):
    tail_ids = _TAIL + lax.broadcasted_iota(jnp.int32, (_R, _NTAIL), 1)
    buf = jnp.concatenate([cv_ref[...], tl_ref[...]], axis=1)
    ibuf = jnp.concatenate([ci_ref[...], tail_ids], axis=1)
    tbuf = jnp.concatenate([ct_ref[...], tt_ref[...]], axis=1)

    picks_v, picks_t = [], []
    for _ in range(_K):
        m = jnp.max(buf, axis=1, keepdims=True)
        ci = jnp.where(buf == m, ibuf, _BIG_I32)
        si = jnp.min(ci, axis=1, keepdims=True)
        sel = ibuf == si
        tm = jnp.sum(jnp.where(sel, tbuf, 0.0), axis=1, keepdims=True)
        picks_v.append(m)
        picks_t.append(tm)
        buf = jnp.where(sel, _NEG_INF, buf)

    vs = jnp.concatenate(picks_v, axis=1)
    ts = jnp.concatenate(picks_t, axis=1)
    probs = jax.nn.sigmoid(vs)
    logp = jnp.maximum(jnp.log(probs), -100.0)
    log1mp = jnp.maximum(jnp.log(1.0 - probs), -100.0)
    row_bce = -jnp.sum(ts * logp + (1.0 - ts) * log1mp, axis=1) / _K
    top_loss = jnp.sum(row_bce) / _R
    bce_mean = bce_ref[0, 0] / (_R * _N)
    out_ref[0, 0] = bce_mean + bcel_ref[0, 0] * top_loss


def _phase_c(cv, ct, ci, tail_l, tail_t, bce_sum, bcel):
    return pl.pallas_call(
        _phase_c_body,
        in_specs=[
            pl.BlockSpec((_R, _CAP), lambda: (0, 0)),
            pl.BlockSpec((_R, _CAP), lambda: (0, 0)),
            pl.BlockSpec((_R, _CAP), lambda: (0, 0)),
            pl.BlockSpec((_R, _NTAIL), lambda: (0, 0)),
            pl.BlockSpec((_R, _NTAIL), lambda: (0, 0)),
            pl.BlockSpec(memory_space=pltpu.SMEM),
            pl.BlockSpec(memory_space=pltpu.SMEM),
        ],
        out_specs=pl.BlockSpec(memory_space=pltpu.SMEM),
        out_shape=jax.ShapeDtypeStruct((1, 1), jnp.float32),
    )(cv, ct, ci, tail_l, tail_t, bce_sum, bcel)


@jax.jit
def kernel(logits, targets, BCE_L):
    bcel = jnp.reshape(BCE_L, (1, 1)).astype(jnp.float32)

    bce_sum, segmax = _phase_a(logits, targets)
    segmax_t = jnp.reshape(jnp.transpose(segmax, (1, 0, 2)), (_R, _NSEGPAD))
    tau = _phase_a2(segmax_t)

    cv, ct, ci = _phase_b(
        logits,
        targets,
        jnp.reshape(tau, (-1,)),
        jnp.reshape(segmax, (-1,)),
    )

    tail_l = lax.slice(logits, (0, _TAIL), (_R, _N))
    tail_t = lax.slice(targets, (0, _TAIL), (_R, _N))

    out = _phase_c(
        jnp.reshape(cv, (_R, _CAP)),
        jnp.reshape(ct, (_R, _CAP)),
        jnp.reshape(ci, (_R, _CAP)),
        tail_l,
        tail_t,
        bce_sum,
        bcel,
    )
    return out[0, 0]
